# baseline (speedup 1.0000x reference)
import jax
import jax.numpy as jnp
from jax.experimental import pallas as pl
from jax.experimental.pallas import tpu as pltpu

_BM = 5000


def _mm_kernel(adj_ref, w_ref, b_ref, out_ref):
    out_ref[...] = (
        jnp.dot(
            adj_ref[...].astype(jnp.bfloat16),
            w_ref[...],
            preferred_element_type=jnp.float32,
        )
        + b_ref[...]
    )


def kernel(adj, weight, bias):
    m, k = adj.shape
    n = weight.shape[1]
    bias2d = bias.reshape(1, n)
    weight_bf16 = weight.astype(jnp.bfloat16)
    return pl.pallas_call(
        _mm_kernel,
        grid=(m // _BM,),
        in_specs=[
            pl.BlockSpec((_BM, k), lambda i: (i, 0)),
            pl.BlockSpec((k, n), lambda i: (0, 0)),
            pl.BlockSpec((1, n), lambda i: (0, 0)),
        ],
        out_specs=pl.BlockSpec((_BM, n), lambda i: (i, 0)),
        out_shape=jax.ShapeDtypeStruct((m, n), jnp.float32),
        compiler_params=pltpu.CompilerParams(
            dimension_semantics=("parallel",),
        ),
    )(adj, weight_bf16, bias2d)


# speedup vs baseline: 1.0206x; 1.0206x over previous
import jax
import jax.numpy as jnp
from jax.experimental import pallas as pl
from jax.experimental.pallas import tpu as pltpu

_BM = 5000


def _mm_kernel(a0_ref, a1_ref, w_ref, b_ref, out_ref):
    acc = jnp.dot(a0_ref[...], w_ref[0], preferred_element_type=jnp.float32)
    acc += jnp.dot(a1_ref[...], w_ref[1], preferred_element_type=jnp.float32)
    out_ref[...] = acc + b_ref[...]


def kernel(adj, weight, bias):
    m, k = adj.shape
    n = weight.shape[1]
    kh = k // 2
    bias2d = bias.reshape(1, n)
    w2 = weight.reshape(2, kh, n)
    return pl.pallas_call(
        _mm_kernel,
        grid=(m // _BM,),
        in_specs=[
            pl.BlockSpec((_BM, kh), lambda i: (i, 0)),
            pl.BlockSpec((_BM, kh), lambda i: (i, 1)),
            pl.BlockSpec((2, kh, n), lambda i: (0, 0, 0)),
            pl.BlockSpec((1, n), lambda i: (0, 0)),
        ],
        out_specs=pl.BlockSpec((_BM, n), lambda i: (i, 0)),
        out_shape=jax.ShapeDtypeStruct((m, n), jnp.float32),
        compiler_params=pltpu.CompilerParams(
            dimension_semantics=("parallel",),
        ),
    )(adj, adj, w2, bias2d)


# speedup vs baseline: 1.0227x; 1.0020x over previous
import jax
import jax.numpy as jnp
from jax.experimental import pallas as pl
from jax.experimental.pallas import tpu as pltpu

_BM = 5000


def _mm_kernel(a0_ref, a1_ref, w_ref, b_ref, out_ref):
    acc = jnp.dot(a0_ref[...], w_ref[0], preferred_element_type=jnp.float32)
    acc += jnp.dot(a1_ref[...], w_ref[1], preferred_element_type=jnp.float32)
    out_ref[...] = acc + b_ref[...]


def kernel(adj, weight, bias):
    m, k = adj.shape
    n = weight.shape[1]
    kh = k // 2
    bias2d = bias.reshape(1, n)
    w2 = weight.reshape(2, kh, n)
    return pl.pallas_call(
        _mm_kernel,
        grid=(m // _BM,),
        in_specs=[
            pl.BlockSpec((_BM, kh), lambda i: (i, 0)),
            pl.BlockSpec((_BM, kh), lambda i: (i, 1)),
            pl.BlockSpec((2, kh, n), lambda i: (0, 0, 0)),
            pl.BlockSpec((1, n), lambda i: (0, 0)),
        ],
        out_specs=pl.BlockSpec((_BM, n), lambda i: (i, 0)),
        out_shape=jax.ShapeDtypeStruct((m, n), jnp.float32),
        compiler_params=pltpu.CompilerParams(
            dimension_semantics=("arbitrary",),
        ),
    )(adj, adj, w2, bias2d)
